# baseline (speedup 1.0000x reference)
import functools

import jax
import jax.numpy as jnp
from jax import lax
from jax.experimental import pallas as pl
from jax.experimental.pallas import tpu as pltpu
from jax.experimental.pallas import tpu_sc as plsc

B, L, D = 4096, 200, 128
N = B * L
NA, NC, NS = 3, 3, 7
NCOMBO = 64


def _combo_body(a_ref, c_ref, s_ref, o_ref):
    r = lax.broadcasted_iota(jnp.int32, (NCOMBO, D), 0)
    ia = r // (NC * NS)
    ic = (r // NS) % NC
    ik = r % NS
    acc = jnp.zeros((NCOMBO, D), jnp.float32)
    for i in range(1, NA):
        acc = acc + jnp.where(ia == i, 1.0, 0.0) * a_ref[i, :]
    for j in range(1, NC):
        acc = acc + jnp.where(ic == j, 1.0, 0.0) * c_ref[j, :]
    for k in range(1, NS):
        acc = acc + jnp.where(ik == k, 1.0, 0.0) * s_ref[k, :]
    o_ref[...] = acc


_combo_call = pl.pallas_call(
    _combo_body,
    out_shape=jax.ShapeDtypeStruct((NCOMBO, D), jnp.float32),
)


def _fuse_body(a_ref, c_ref, s_ref, o_ref):
    o_ref[...] = a_ref[...] * (NC * NS) + c_ref[...] * NS + s_ref[...]


_FB = 128
_fuse_call = pl.pallas_call(
    _fuse_body,
    grid=(B // _FB,),
    in_specs=[pl.BlockSpec((_FB, L), lambda i: (i, 0))] * 3,
    out_specs=pl.BlockSpec((_FB, L), lambda i: (i, 0)),
    out_shape=jax.ShapeDtypeStruct((B, L), jnp.int32),
)


_NCORES = 2
_NSUB = 16
_NW = _NCORES * _NSUB
_LANES = 16
CHUNK = 128
ROWS_PW = N // _NW
NCHUNK = ROWS_PW // CHUNK

NBUF = 4
NSUPER = NCHUNK // NBUF


@functools.lru_cache(maxsize=1)
def _get_sc_embed():
    mesh = plsc.VectorSubcoreMesh(core_axis_name="c", subcore_axis_name="s")

    @functools.partial(
        pl.kernel,
        mesh=mesh,
        out_type=jax.ShapeDtypeStruct((N, D), jnp.float32),
        scratch_types=[
            pltpu.VMEM((NBUF, CHUNK), jnp.int32),
            pltpu.VMEM((NBUF, CHUNK, D), jnp.float32),
            pltpu.VMEM_SHARED((NCOMBO, D), jnp.float32),
        ] + [pltpu.SemaphoreType.DMA] * (3 * NBUF),
    )
    def _sc_embed(fx_hbm, combo_hbm, out_hbm,
                  fx_v, rows_v, combo_sh, *sems):
        isem = sems[0:NBUF]
        gsem = sems[NBUF:2 * NBUF]
        ssem = sems[2 * NBUF:3 * NBUF]
        wid = lax.axis_index("s") * _NCORES + lax.axis_index("c")
        w_base = wid * ROWS_PW

        def start_idx(g, b):
            base = w_base + g * CHUNK
            pltpu.async_copy(fx_hbm.at[pl.ds(base, CHUNK)], fx_v.at[b],
                             isem[b])

        def wait_idx(b):
            pltpu.make_async_copy(fx_hbm.at[pl.ds(0, CHUNK)], fx_v.at[b],
                                  isem[b]).wait()

        def start_gather(b):
            pltpu.async_copy(combo_sh.at[fx_v.at[b]], rows_v.at[b], gsem[b])

        def wait_gather(b):
            pltpu.make_async_copy(out_hbm.at[pl.ds(0, CHUNK)], rows_v.at[b],
                                  gsem[b]).wait()

        def start_store(g, b):
            base = w_base + g * CHUNK
            pltpu.async_copy(rows_v.at[b], out_hbm.at[pl.ds(base, CHUNK)],
                             ssem[b])

        def wait_store(b):
            pltpu.make_async_copy(rows_v.at[b], out_hbm.at[pl.ds(0, CHUNK)],
                                  ssem[b]).wait()

        @pl.when(lax.axis_index("s") == 0)
        def _():
            pltpu.sync_copy(combo_hbm, combo_sh)
        plsc.subcore_barrier()

        for b in range(NBUF):
            start_idx(b, b)
        for b in range(NBUF):
            wait_idx(b)
            start_gather(b)
            if b > 0:
                wait_gather(b - 1)
                start_store(b - 1, b - 1)
                start_idx(b - 1 + NBUF, b - 1)

        def super_body(it, carry):
            g0 = it * NBUF
            for b in range(NBUF):
                g = g0 + b
                wait_idx(b)
                wait_store(b)
                start_gather(b)

                b1 = (b - 1) % NBUF
                wait_gather(b1)
                start_store(g - 1, b1)

                @pl.when(g - 1 + NBUF < NCHUNK)
                def _():
                    start_idx(g - 1 + NBUF, b1)
            return carry

        lax.fori_loop(1, NSUPER, super_body, 0)

        last = NCHUNK - 1
        wait_gather(last % NBUF)
        start_store(last, last % NBUF)
        for b in range(NBUF):
            wait_store(b)

    return _sc_embed


def kernel(prop_bond_aromatic, prop_bond_conjugated, prop_bond_stereo,
           aromatic_table, conjugated_table, stereo_table):
    combo = _combo_call(aromatic_table, conjugated_table, stereo_table)
    fused2d = _fuse_call(prop_bond_aromatic.astype(jnp.int32),
                         prop_bond_conjugated.astype(jnp.int32),
                         prop_bond_stereo.astype(jnp.int32))
    fx = fused2d.reshape(N)
    out = _get_sc_embed()(fx, combo)
    return out.reshape(B, L, D)


# speedup vs baseline: 1.0320x; 1.0320x over previous
import functools

import jax
import jax.numpy as jnp
from jax import lax
from jax.experimental import pallas as pl
from jax.experimental.pallas import tpu as pltpu
from jax.experimental.pallas import tpu_sc as plsc

B, L, D = 4096, 200, 128
N = B * L
NA, NC, NS = 3, 3, 7
NCOMBO = 64


def _combo_body(a_ref, c_ref, s_ref, o_ref):
    r = lax.broadcasted_iota(jnp.int32, (NCOMBO, D), 0)
    ia = r // (NC * NS)
    ic = (r // NS) % NC
    ik = r % NS
    acc = jnp.zeros((NCOMBO, D), jnp.float32)
    for i in range(1, NA):
        acc = acc + jnp.where(ia == i, 1.0, 0.0) * a_ref[i, :]
    for j in range(1, NC):
        acc = acc + jnp.where(ic == j, 1.0, 0.0) * c_ref[j, :]
    for k in range(1, NS):
        acc = acc + jnp.where(ik == k, 1.0, 0.0) * s_ref[k, :]
    o_ref[...] = acc


_combo_call = pl.pallas_call(
    _combo_body,
    out_shape=jax.ShapeDtypeStruct((NCOMBO, D), jnp.float32),
)


_NCORES = 2
_NSUB = 16
_NW = _NCORES * _NSUB
_LANES = 16
CHUNK = 128
ROWS_PW = N // _NW
NCHUNK = ROWS_PW // CHUNK

NBUF = 5
NSUPER = NCHUNK // NBUF


@functools.lru_cache(maxsize=1)
def _get_sc_embed():
    mesh = plsc.VectorSubcoreMesh(core_axis_name="c", subcore_axis_name="s")

    @functools.partial(
        pl.kernel,
        mesh=mesh,
        out_type=jax.ShapeDtypeStruct((N, D), jnp.float32),
        scratch_types=[
            pltpu.VMEM((NBUF, CHUNK), jnp.int32),
            pltpu.VMEM((NBUF, CHUNK), jnp.int32),
            pltpu.VMEM((NBUF, CHUNK), jnp.int32),
            pltpu.VMEM((NBUF, CHUNK), jnp.int32),
            pltpu.VMEM((NBUF, CHUNK, D), jnp.float32),
            pltpu.VMEM_SHARED((NCOMBO, D), jnp.float32),
        ] + [pltpu.SemaphoreType.DMA] * (3 * NBUF),
    )
    def _sc_embed(ia_hbm, ic_hbm, is_hbm, combo_hbm, out_hbm,
                  ia_v, ic_v, is_v, fx_v, rows_v, combo_sh, *sems):
        isem = sems[0:NBUF]
        gsem = sems[NBUF:2 * NBUF]
        ssem = sems[2 * NBUF:3 * NBUF]
        wid = lax.axis_index("s") * _NCORES + lax.axis_index("c")
        w_base = wid * ROWS_PW
        idx_pairs = ((ia_hbm, ia_v), (ic_hbm, ic_v), (is_hbm, is_v))

        def start_idx(g, b):
            base = w_base + g * CHUNK
            for h, v in idx_pairs:
                pltpu.async_copy(h.at[pl.ds(base, CHUNK)], v.at[b], isem[b])

        def wait_idx(b):
            for h, v in idx_pairs:
                pltpu.make_async_copy(h.at[pl.ds(0, CHUNK)], v.at[b],
                                      isem[b]).wait()

        def compute_fused(b):
            for i in range(CHUNK // _LANES):
                sl = pl.ds(i * _LANES, _LANES)
                fx_v[b, sl] = (ia_v[b, sl] * (NC * NS)
                               + ic_v[b, sl] * NS + is_v[b, sl])

        def start_gather(b):
            pltpu.async_copy(combo_sh.at[fx_v.at[b]], rows_v.at[b], gsem[b])

        def wait_gather(b):
            pltpu.make_async_copy(out_hbm.at[pl.ds(0, CHUNK)], rows_v.at[b],
                                  gsem[b]).wait()

        def start_store(g, b):
            base = w_base + g * CHUNK
            pltpu.async_copy(rows_v.at[b], out_hbm.at[pl.ds(base, CHUNK)],
                             ssem[b])

        def wait_store(b):
            pltpu.make_async_copy(rows_v.at[b], out_hbm.at[pl.ds(0, CHUNK)],
                                  ssem[b]).wait()

        @pl.when(lax.axis_index("s") == 0)
        def _():
            pltpu.sync_copy(combo_hbm, combo_sh)
        plsc.subcore_barrier()

        for b in range(NBUF):
            start_idx(b, b)
        for b in range(NBUF):
            wait_idx(b)
            compute_fused(b)
            start_gather(b)
            start_idx(b + NBUF, b)
            if b > 0:
                wait_gather(b - 1)
                start_store(b - 1, b - 1)

        def super_body(it, carry):
            g0 = it * NBUF
            for b in range(NBUF):
                g = g0 + b
                wait_idx(b)
                compute_fused(b)
                wait_store(b)
                start_gather(b)

                @pl.when(g + NBUF < NCHUNK)
                def _():
                    start_idx(g + NBUF, b)

                b1 = (b - 1) % NBUF
                wait_gather(b1)
                start_store(g - 1, b1)
            return carry

        lax.fori_loop(1, NSUPER, super_body, 0)

        last = NCHUNK - 1
        wait_gather(last % NBUF)
        start_store(last, last % NBUF)
        for b in range(NBUF):
            wait_store(b)

    return _sc_embed


def kernel(prop_bond_aromatic, prop_bond_conjugated, prop_bond_stereo,
           aromatic_table, conjugated_table, stereo_table):
    combo = _combo_call(aromatic_table, conjugated_table, stereo_table)
    ia = prop_bond_aromatic.reshape(N).astype(jnp.int32)
    ic = prop_bond_conjugated.reshape(N).astype(jnp.int32)
    ik = prop_bond_stereo.reshape(N).astype(jnp.int32)
    out = _get_sc_embed()(ia, ic, ik, combo)
    return out.reshape(B, L, D)


# speedup vs baseline: 1.0470x; 1.0146x over previous
import functools

import jax
import jax.numpy as jnp
from jax import lax
from jax.experimental import pallas as pl
from jax.experimental.pallas import tpu as pltpu
from jax.experimental.pallas import tpu_sc as plsc

B, L, D = 4096, 200, 128
N = B * L
NA, NC, NS = 3, 3, 7
NCOMBO = 64


def _combo_body(a_ref, c_ref, s_ref, o_ref):
    r = lax.broadcasted_iota(jnp.int32, (NCOMBO, D), 0)
    ia = r // (NC * NS)
    ic = (r // NS) % NC
    ik = r % NS
    acc = jnp.zeros((NCOMBO, D), jnp.float32)
    for i in range(1, NA):
        acc = acc + jnp.where(ia == i, 1.0, 0.0) * a_ref[i, :]
    for j in range(1, NC):
        acc = acc + jnp.where(ic == j, 1.0, 0.0) * c_ref[j, :]
    for k in range(1, NS):
        acc = acc + jnp.where(ik == k, 1.0, 0.0) * s_ref[k, :]
    o_ref[...] = acc


_combo_call = pl.pallas_call(
    _combo_body,
    out_shape=jax.ShapeDtypeStruct((NCOMBO, D), jnp.float32),
)


_NCORES = 2
_NSUB = 16
_NW = _NCORES * _NSUB
_LANES = 16

ROWS_PW = B // _NW
GRP = 8
NGRP = ROWS_PW // GRP
LA = 128
LB = L - LA
RW = 2 * LA
NBUF = 4


@functools.lru_cache(maxsize=1)
def _get_sc_embed():
    mesh = plsc.VectorSubcoreMesh(core_axis_name="c", subcore_axis_name="s")

    @functools.partial(
        pl.kernel,
        mesh=mesh,
        out_type=jax.ShapeDtypeStruct((N, D), jnp.float32),
        compiler_params=pltpu.CompilerParams(use_tc_tiling_on_sc=True),
        scratch_types=[
            pltpu.VMEM((2, GRP, LA), jnp.int32),
            pltpu.VMEM((2, GRP, LA), jnp.int32),
            pltpu.VMEM((2, GRP, LA), jnp.int32),
            pltpu.VMEM((2, GRP, LA), jnp.int32),
            pltpu.VMEM((2, GRP, LA), jnp.int32),
            pltpu.VMEM((2, GRP, LA), jnp.int32),
            pltpu.VMEM((2 * GRP * RW,), jnp.int32),
            pltpu.VMEM((NBUF, LA, D), jnp.float32),
            pltpu.VMEM((NBUF, LB, D), jnp.float32),
            pltpu.VMEM_SHARED((NCOMBO, D), jnp.float32),
        ] + [pltpu.SemaphoreType.DMA] * (2 + 4 * NBUF),
    )
    def _sc_embed(ia_hbm, ic_hbm, is_hbm, iat_hbm, ict_hbm, ist_hbm,
                  combo_hbm, out_hbm,
                  iaA, iaT, icA, icT, isA, isT, fx,
                  rowsA, rowsB, combo_sh, *sems):
        isem = sems[0:2]
        gA = sems[2:2 + NBUF]
        gB = sems[2 + NBUF:2 + 2 * NBUF]
        sA = sems[2 + 2 * NBUF:2 + 3 * NBUF]
        sB = sems[2 + 3 * NBUF:2 + 4 * NBUF]
        wid = lax.axis_index("s") * _NCORES + lax.axis_index("c")
        w_row0 = wid * ROWS_PW
        idx_trip = ((ia_hbm, iat_hbm, iaA, iaT),
                    (ic_hbm, ict_hbm, icA, icT),
                    (is_hbm, ist_hbm, isA, isT))

        def start_idx(g, s):
            r0 = pl.multiple_of(w_row0 + g * GRP, GRP)
            for h, ht, vA, vT in idx_trip:
                pltpu.async_copy(h.at[pl.ds(r0, GRP), pl.ds(0, LA)],
                                 vA.at[s], isem[s])
                pltpu.async_copy(ht.at[pl.ds(r0, GRP)], vT.at[s], isem[s])

        def wait_idx(s):
            for h, ht, vA, vT in idx_trip:
                pltpu.make_async_copy(h.at[pl.ds(0, GRP), pl.ds(0, LA)],
                                      vA.at[s], isem[s]).wait()
                pltpu.make_async_copy(ht.at[pl.ds(0, GRP)], vT.at[s],
                                      isem[s]).wait()

        def compute_fused(s):
            base = s * GRP * RW
            for r in range(GRP):
                for j in range(LA // _LANES):
                    sl = pl.ds(j * _LANES, _LANES)
                    fx[pl.ds(base + r * RW + j * _LANES, _LANES)] = (
                        iaA[s, r, sl] * (NC * NS)
                        + icA[s, r, sl] * NS + isA[s, r, sl])
                for j in range(5):
                    sl = pl.ds(j * _LANES, _LANES)
                    fx[pl.ds(base + r * RW + LA + j * _LANES, _LANES)] = (
                        iaT[s, r, sl] * (NC * NS)
                        + icT[s, r, sl] * NS + isT[s, r, sl])

        def start_gathers(s, r):
            b = r % NBUF
            base = s * GRP * RW + r * RW
            pltpu.async_copy(combo_sh.at[fx.at[pl.ds(base, LA)]],
                             rowsA.at[b], gA[b])
            pltpu.async_copy(combo_sh.at[fx.at[pl.ds(base + LA, LB)]],
                             rowsB.at[b], gB[b])

        def wait_gathers(b):
            pltpu.make_async_copy(out_hbm.at[pl.ds(0, LA)], rowsA.at[b],
                                  gA[b]).wait()
            pltpu.make_async_copy(out_hbm.at[pl.ds(0, LB)], rowsB.at[b],
                                  gB[b]).wait()

        def start_stores(row, b):
            pltpu.async_copy(rowsA.at[b],
                             out_hbm.at[pl.ds(pl.multiple_of(row * L, 8),
                                              LA)], sA[b])
            pltpu.async_copy(rowsB.at[b],
                             out_hbm.at[pl.ds(pl.multiple_of(row * L + LA, 8),
                                              LB)], sB[b])

        def wait_stores(b):
            pltpu.make_async_copy(rowsA.at[b], out_hbm.at[pl.ds(0, LA)],
                                  sA[b]).wait()
            pltpu.make_async_copy(rowsB.at[b], out_hbm.at[pl.ds(0, LB)],
                                  sB[b]).wait()

        @pl.when(lax.axis_index("s") == 0)
        def _():
            pltpu.sync_copy(combo_hbm, combo_sh)
        plsc.subcore_barrier()

        def group_body(g, s, first):
            wait_idx(s)
            compute_fused(s)

            @pl.when(g + 1 < NGRP)
            def _():
                start_idx(g + 1, 1 - s)

            for r in range(GRP):
                b = r % NBUF
                if not (first and r < NBUF):
                    wait_stores(b)
                start_gathers(s, r)
                if not (first and r == 0):
                    bp = (r - 1) % NBUF
                    wait_gathers(bp)
                    start_stores(w_row0 + g * GRP + (r - 1), bp)

        start_idx(0, 0)
        group_body(0, 0, True)
        group_body(1, 1, False)

        def pair_body(k, carry):
            g = 2 * k
            group_body(g, 0, False)
            group_body(g + 1, 1, False)
            return carry

        lax.fori_loop(1, NGRP // 2, pair_body, 0)

        lastb = (GRP - 1) % NBUF
        wait_gathers(lastb)
        start_stores(w_row0 + ROWS_PW - 1, lastb)
        for b in range(NBUF):
            wait_stores(b)

    return _sc_embed


def kernel(prop_bond_aromatic, prop_bond_conjugated, prop_bond_stereo,
           aromatic_table, conjugated_table, stereo_table):
    combo = _combo_call(aromatic_table, conjugated_table, stereo_table)
    ia = prop_bond_aromatic.astype(jnp.int32)
    ic = prop_bond_conjugated.astype(jnp.int32)
    ik = prop_bond_stereo.astype(jnp.int32)
    pad = ((0, 0), (0, LA - LB))
    iat = jnp.pad(ia[:, LA:], pad)
    ict = jnp.pad(ic[:, LA:], pad)
    ist = jnp.pad(ik[:, LA:], pad)
    out = _get_sc_embed()(ia, ic, ik, iat, ict, ist, combo)
    return out.reshape(B, L, D)


# speedup vs baseline: 1.0489x; 1.0018x over previous
import functools

import jax
import jax.numpy as jnp
from jax import lax
from jax.experimental import pallas as pl
from jax.experimental.pallas import tpu as pltpu
from jax.experimental.pallas import tpu_sc as plsc

B, L, D = 4096, 200, 128
N = B * L
NA, NC, NS = 3, 3, 7
NCOMBO = 64


def _combo_body(a_ref, c_ref, s_ref, o_ref):
    r = lax.broadcasted_iota(jnp.int32, (NCOMBO, D), 0)
    ia = r // (NC * NS)
    ic = (r // NS) % NC
    ik = r % NS
    acc = jnp.zeros((NCOMBO, D), jnp.float32)
    for i in range(1, NA):
        acc = acc + jnp.where(ia == i, 1.0, 0.0) * a_ref[i, :]
    for j in range(1, NC):
        acc = acc + jnp.where(ic == j, 1.0, 0.0) * c_ref[j, :]
    for k in range(1, NS):
        acc = acc + jnp.where(ik == k, 1.0, 0.0) * s_ref[k, :]
    o_ref[...] = acc


_combo_call = pl.pallas_call(
    _combo_body,
    out_shape=jax.ShapeDtypeStruct((NCOMBO, D), jnp.float32),
)


_NCORES = 2
_NSUB = 16
_NW = _NCORES * _NSUB
_LANES = 16

ROWS_PW = B // _NW
GRP = 8
NGRP = ROWS_PW // GRP
LA = 128
LB = L - LA
RW = 2 * LA
NBUF = 4


@functools.lru_cache(maxsize=1)
def _get_sc_embed():
    mesh = plsc.VectorSubcoreMesh(core_axis_name="c", subcore_axis_name="s")

    @functools.partial(
        pl.kernel,
        mesh=mesh,
        out_type=jax.ShapeDtypeStruct((N, D), jnp.float32),
        compiler_params=pltpu.CompilerParams(use_tc_tiling_on_sc=True),
        scratch_types=[
            pltpu.VMEM((2, GRP, LA), jnp.int32),
            pltpu.VMEM((2, GRP, LA), jnp.int32),
            pltpu.VMEM((2, GRP, LA), jnp.int32),
            pltpu.VMEM((2, GRP, LA), jnp.int32),
            pltpu.VMEM((2, GRP, LA), jnp.int32),
            pltpu.VMEM((2, GRP, LA), jnp.int32),
            pltpu.VMEM((2 * GRP * RW,), jnp.int32),
            pltpu.VMEM((NBUF, L, D), jnp.float32),
            pltpu.VMEM_SHARED((NCOMBO, D), jnp.float32),
        ] + [pltpu.SemaphoreType.DMA] * (2 + 3 * NBUF),
    )
    def _sc_embed(ia_hbm, ic_hbm, is_hbm, iat_hbm, ict_hbm, ist_hbm,
                  combo_hbm, out_hbm,
                  iaA, iaT, icA, icT, isA, isT, fx,
                  rows, combo_sh, *sems):
        isem = sems[0:2]
        gA = sems[2:2 + NBUF]
        gB = sems[2 + NBUF:2 + 2 * NBUF]
        sS = sems[2 + 2 * NBUF:2 + 3 * NBUF]
        wid = lax.axis_index("s") * _NCORES + lax.axis_index("c")
        w_row0 = wid * ROWS_PW
        idx_trip = ((ia_hbm, iat_hbm, iaA, iaT),
                    (ic_hbm, ict_hbm, icA, icT),
                    (is_hbm, ist_hbm, isA, isT))

        def start_idx(g, s):
            r0 = pl.multiple_of(w_row0 + g * GRP, GRP)
            for h, ht, vA, vT in idx_trip:
                pltpu.async_copy(h.at[pl.ds(r0, GRP), pl.ds(0, LA)],
                                 vA.at[s], isem[s])
                pltpu.async_copy(ht.at[pl.ds(r0, GRP)], vT.at[s], isem[s])

        def wait_idx(s):
            for h, ht, vA, vT in idx_trip:
                pltpu.make_async_copy(h.at[pl.ds(0, GRP), pl.ds(0, LA)],
                                      vA.at[s], isem[s]).wait()
                pltpu.make_async_copy(ht.at[pl.ds(0, GRP)], vT.at[s],
                                      isem[s]).wait()

        def compute_fused(s):
            base = s * GRP * RW
            for r in range(GRP):
                for j in range(LA // _LANES):
                    sl = pl.ds(j * _LANES, _LANES)
                    fx[pl.ds(base + r * RW + j * _LANES, _LANES)] = (
                        iaA[s, r, sl] * (NC * NS)
                        + icA[s, r, sl] * NS + isA[s, r, sl])
                for j in range(5):
                    sl = pl.ds(j * _LANES, _LANES)
                    fx[pl.ds(base + r * RW + LA + j * _LANES, _LANES)] = (
                        iaT[s, r, sl] * (NC * NS)
                        + icT[s, r, sl] * NS + isT[s, r, sl])

        def start_gathers(s, r):
            b = r % NBUF
            base = s * GRP * RW + r * RW
            pltpu.async_copy(combo_sh.at[fx.at[pl.ds(base, LA)]],
                             rows.at[b, pl.ds(0, LA)], gA[b])
            pltpu.async_copy(combo_sh.at[fx.at[pl.ds(base + LA, LB)]],
                             rows.at[b, pl.ds(LA, LB)], gB[b])

        def wait_gathers(b):
            pltpu.make_async_copy(out_hbm.at[pl.ds(0, LA)],
                                  rows.at[b, pl.ds(0, LA)], gA[b]).wait()
            pltpu.make_async_copy(out_hbm.at[pl.ds(0, LB)],
                                  rows.at[b, pl.ds(LA, LB)], gB[b]).wait()

        def start_stores(row, b):
            pltpu.async_copy(rows.at[b],
                             out_hbm.at[pl.ds(pl.multiple_of(row * L, 8), L)],
                             sS[b])

        def wait_stores(b):
            pltpu.make_async_copy(rows.at[b], out_hbm.at[pl.ds(0, L)],
                                  sS[b]).wait()

        @pl.when(lax.axis_index("s") == 0)
        def _():
            pltpu.sync_copy(combo_hbm, combo_sh)
        plsc.subcore_barrier()

        def group_body(g, s, first):
            wait_idx(s)
            compute_fused(s)

            @pl.when(g + 1 < NGRP)
            def _():
                start_idx(g + 1, 1 - s)

            for r in range(GRP):
                b = r % NBUF
                if not (first and r < NBUF):
                    wait_stores(b)
                start_gathers(s, r)
                if not (first and r == 0):
                    bp = (r - 1) % NBUF
                    wait_gathers(bp)
                    start_stores(w_row0 + g * GRP + (r - 1), bp)

        start_idx(0, 0)
        group_body(0, 0, True)
        group_body(1, 1, False)

        def pair_body(k, carry):
            g = 2 * k
            group_body(g, 0, False)
            group_body(g + 1, 1, False)
            return carry

        lax.fori_loop(1, NGRP // 2, pair_body, 0)

        lastb = (GRP - 1) % NBUF
        wait_gathers(lastb)
        start_stores(w_row0 + ROWS_PW - 1, lastb)
        for b in range(NBUF):
            wait_stores(b)

    return _sc_embed


def kernel(prop_bond_aromatic, prop_bond_conjugated, prop_bond_stereo,
           aromatic_table, conjugated_table, stereo_table):
    combo = _combo_call(aromatic_table, conjugated_table, stereo_table)
    ia = prop_bond_aromatic.astype(jnp.int32)
    ic = prop_bond_conjugated.astype(jnp.int32)
    ik = prop_bond_stereo.astype(jnp.int32)
    pad = ((0, 0), (0, LA - LB))
    iat = jnp.pad(ia[:, LA:], pad)
    ict = jnp.pad(ic[:, LA:], pad)
    ist = jnp.pad(ik[:, LA:], pad)
    out = _get_sc_embed()(ia, ic, ik, iat, ict, ist, combo)
    return out.reshape(B, L, D)
